# baseline (speedup 1.0000x reference)
import functools

import jax
import jax.numpy as jnp
from jax import lax
from jax.experimental import pallas as pl
from jax.experimental.pallas import tpu as pltpu
from jax.experimental.pallas import tpu_sc as plsc

_B = 16384
_D = 64
_H = 128

_info = plsc.get_sparse_core_info()
_NC, _NS = _info.num_cores, _info.num_subcores
_NW = _NC * _NS
_BPW = _B // _NW
_CHUNK = 128
_NCH = _BPW // _CHUNK

_mesh = plsc.VectorSubcoreMesh(core_axis_name="c", subcore_axis_name="s")


@functools.partial(
    pl.kernel,
    out_type=[
        jax.ShapeDtypeStruct((_B, _D), jnp.float32),
        jax.ShapeDtypeStruct((_B, _D), jnp.float32),
    ],
    mesh=_mesh,
    compiler_params=pltpu.CompilerParams(use_tc_tiling_on_sc=False),
    scratch_types=[
        pltpu.VMEM((_NCH, _CHUNK), jnp.int32),
        pltpu.VMEM((_NCH, _CHUNK), jnp.int32),
        pltpu.VMEM((_BPW, _D), jnp.float32),
        pltpu.VMEM((_BPW, _D), jnp.float32),
        pltpu.SemaphoreType.DMA,
        pltpu.SemaphoreType.DMA,
    ],
)
def _sc_gather(users_hbm, items_hbm, utab_hbm, itab_hbm, uout_hbm, iout_hbm,
               uidx_v, iidx_v, urows_v, irows_v, sem_u, sem_i):
    wid = lax.axis_index("s") * _NC + lax.axis_index("c")
    base = wid * _BPW
    pltpu.sync_copy(users_hbm.at[wid], uidx_v)
    pltpu.sync_copy(items_hbm.at[wid], iidx_v)
    copies = []
    for j in range(_NCH):
        dst = slice(j * _CHUNK, (j + 1) * _CHUNK)
        copies.append(pltpu.async_copy(
            utab_hbm.at[uidx_v.at[j]], urows_v.at[dst], sem_u))
        copies.append(pltpu.async_copy(
            itab_hbm.at[iidx_v.at[j]], irows_v.at[dst], sem_i))
    for c in copies:
        c.wait()
    pltpu.sync_copy(urows_v, uout_hbm.at[pl.ds(base, _BPW)])
    pltpu.sync_copy(irows_v, iout_hbm.at[pl.ds(base, _BPW)])


_BLK = 2048


def _mlp_body(u_ref, i_ref, w1_ref, b1_ref, w2_ref, b2_ref, o_ref):
    w1 = w1_ref[...]
    h = lax.dot_general(u_ref[...], w1[:, :_D], (((1,), (1,)), ((), ())),
                        preferred_element_type=jnp.float32,
                        precision=lax.Precision.HIGHEST)
    h += lax.dot_general(i_ref[...], w1[:, _D:], (((1,), (1,)), ((), ())),
                         preferred_element_type=jnp.float32,
                         precision=lax.Precision.HIGHEST)
    h += b1_ref[...]
    h = jnp.maximum(h, 0.0)
    o_ref[...] = jnp.sum(h * w2_ref[...], axis=1) + b2_ref[0, 0]


_mlp = pl.pallas_call(
    _mlp_body,
    grid=(_B // _BLK,),
    in_specs=[
        pl.BlockSpec((_BLK, _D), lambda b: (b, 0)),
        pl.BlockSpec((_BLK, _D), lambda b: (b, 0)),
        pl.BlockSpec((_H, 2 * _D), lambda b: (0, 0)),
        pl.BlockSpec((1, _H), lambda b: (0, 0)),
        pl.BlockSpec((1, _H), lambda b: (0, 0)),
        pl.BlockSpec((1, 1), lambda b: (0, 0)),
    ],
    out_specs=pl.BlockSpec((_BLK,), lambda b: (b,)),
    out_shape=jax.ShapeDtypeStruct((_B,), jnp.float32),
)


def kernel(users, items, user_table, item_table, w1, b1, w2, b2):
    users_r = users.astype(jnp.int32).reshape(_NW, _NCH, _CHUNK)
    items_r = items.astype(jnp.int32).reshape(_NW, _NCH, _CHUNK)
    u_emb, i_emb = _sc_gather(users_r, items_r, user_table, item_table)
    return _mlp(u_emb, i_emb, w1, b1.reshape(1, _H), w2.reshape(1, _H),
                b2.reshape(1, 1))


# speedup vs baseline: 1.0104x; 1.0104x over previous
import functools

import jax
import jax.numpy as jnp
from jax import lax
from jax.experimental import pallas as pl
from jax.experimental.pallas import tpu as pltpu
from jax.experimental.pallas import tpu_sc as plsc

_B = 16384
_D = 64
_H = 128
_PW = 2 * _D

_info = plsc.get_sparse_core_info()
_NC, _NS = _info.num_cores, _info.num_subcores
_NW = _NC * _NS
_BPW = _B // _NW
_CHUNK = 128
_NCH = _BPW // _CHUNK

_mesh = plsc.VectorSubcoreMesh(core_axis_name="c", subcore_axis_name="s")


@functools.partial(
    pl.kernel,
    out_type=[
        jax.ShapeDtypeStruct((_B, _PW), jnp.float32),
        jax.ShapeDtypeStruct((_B, _PW), jnp.float32),
    ],
    mesh=_mesh,
    scratch_types=[
        pltpu.VMEM((_BPW,), jnp.int32),
        pltpu.VMEM((_BPW,), jnp.int32),
        pltpu.VMEM((_BPW, _PW), jnp.float32),
        pltpu.SemaphoreType.DMA,
    ],
)
def _sc_gather(users_hbm, items_hbm, utab_hbm, itab_hbm, uout_hbm, iout_hbm,
               uidx_v, iidx_v, rows_v, sem):
    wid = lax.axis_index("s") * _NC + lax.axis_index("c")
    base = wid * _BPW
    pltpu.sync_copy(users_hbm.at[pl.ds(base, _BPW)], uidx_v)
    pltpu.sync_copy(items_hbm.at[pl.ds(base, _BPW)], iidx_v)
    for idx_v, tab, out in ((uidx_v, utab_hbm, uout_hbm),
                            (iidx_v, itab_hbm, iout_hbm)):
        copies = []
        for j in range(_NCH):
            sl = pl.ds(j * _CHUNK, _CHUNK)
            copies.append(pltpu.async_copy(
                tab.at[idx_v.at[sl]], rows_v.at[sl], sem))
        for c in copies:
            c.wait()
        pltpu.sync_copy(rows_v, out.at[pl.ds(base, _BPW)])


_BLK = 2048


def _mlp_body(urow_ref, irow_ref, upar_ref, ipar_ref,
              w1_ref, b1_ref, w2_ref, b2_ref, o_ref):
    urow = urow_ref[...]
    irow = irow_ref[...]
    u = jnp.where(upar_ref[...] > 0, urow[:, _D:], urow[:, :_D])
    i = jnp.where(ipar_ref[...] > 0, irow[:, _D:], irow[:, :_D])
    w1 = w1_ref[...]
    h = lax.dot_general(u, w1[:, :_D], (((1,), (1,)), ((), ())),
                        preferred_element_type=jnp.float32,
                        precision=lax.Precision.HIGHEST)
    h += lax.dot_general(i, w1[:, _D:], (((1,), (1,)), ((), ())),
                         preferred_element_type=jnp.float32,
                         precision=lax.Precision.HIGHEST)
    h += b1_ref[...]
    h = jnp.maximum(h, 0.0)
    o_ref[...] = jnp.sum(h * w2_ref[...], axis=1) + b2_ref[0, 0]


_mlp = pl.pallas_call(
    _mlp_body,
    grid=(_B // _BLK,),
    in_specs=[
        pl.BlockSpec((_BLK, _PW), lambda b: (b, 0)),
        pl.BlockSpec((_BLK, _PW), lambda b: (b, 0)),
        pl.BlockSpec((_BLK, 1), lambda b: (b, 0)),
        pl.BlockSpec((_BLK, 1), lambda b: (b, 0)),
        pl.BlockSpec((_H, 2 * _D), lambda b: (0, 0)),
        pl.BlockSpec((1, _H), lambda b: (0, 0)),
        pl.BlockSpec((1, _H), lambda b: (0, 0)),
        pl.BlockSpec((1, 1), lambda b: (0, 0)),
    ],
    out_specs=pl.BlockSpec((_BLK,), lambda b: (b,)),
    out_shape=jax.ShapeDtypeStruct((_B,), jnp.float32),
)


def kernel(users, items, user_table, item_table, w1, b1, w2, b2):
    users = users.astype(jnp.int32)
    items = items.astype(jnp.int32)
    utab2 = user_table.reshape(-1, _PW)
    itab2 = item_table.reshape(-1, _PW)
    urows, irows = _sc_gather(users >> 1, items >> 1, utab2, itab2)
    upar = (users & 1).reshape(_B, 1)
    ipar = (items & 1).reshape(_B, 1)
    return _mlp(urows, irows, upar, ipar, w1,
                b1.reshape(1, _H), w2.reshape(1, _H), b2.reshape(1, 1))


# speedup vs baseline: 2.1235x; 2.1017x over previous
import functools

import jax
import jax.numpy as jnp
from jax import lax
from jax.experimental import pallas as pl
from jax.experimental.pallas import tpu as pltpu
from jax.experimental.pallas import tpu_sc as plsc

_B = 16384
_D = 64
_H = 128
_V = 1000000

_info = plsc.get_sparse_core_info()
_NC, _NS = _info.num_cores, _info.num_subcores
_NW = _NC * _NS
_SLAB = 512
_NSLAB_T = (_V // _SLAB) // _NW
_REG = _NSLAB_T * _SLAB
_LAST_FULL = _NW * _REG
_TAIL_LO = _V - 128
_LISTCAP = _B + 16

_mesh = plsc.VectorSubcoreMesh(core_axis_name="c", subcore_axis_name="s")


@functools.partial(
    pl.kernel,
    out_type=[
        jax.ShapeDtypeStruct((_B + 16, 2 * _D), jnp.float32),
        jax.ShapeDtypeStruct((_B + 16, 2 * _D), jnp.float32),
    ],
    mesh=_mesh,
    compiler_params=pltpu.CompilerParams(needs_layout_passes=False),
    scratch_types=[
        pltpu.VMEM((_B,), jnp.int32),
        pltpu.VMEM((_LISTCAP,), jnp.int32),
        pltpu.VMEM((_LISTCAP,), jnp.int32),
        pltpu.VMEM((32,), jnp.int32),
        pltpu.VMEM((32,), jnp.int32),
        pltpu.VMEM((_D, _SLAB), jnp.float32),
        pltpu.VMEM((_D, _SLAB), jnp.float32),
        pltpu.VMEM((32, 2 * _D), jnp.float32),
        pltpu.VMEM((2, 16), jnp.int32),
        pltpu.SemaphoreType.DMA,
        pltpu.SemaphoreType.DMA,
        pltpu.SemaphoreType.DMA,
    ],
)
def _sc_gather(users_hbm, items_hbm, utabT_hbm, itabT_hbm,
               utail_hbm, itail_hbm, uout_hbm, iout_hbm,
               idx_v, list_r, list_p, ring_r, ring_p,
               slab0_v, slab1_v, stage_v, stagepos_v,
               sem_b0, sem_b1, sem_sc):
    wid = lax.axis_index("s") * _NC + lax.axis_index("c")
    lanes = lax.iota(jnp.int32, 16)
    lo = wid * _REG
    hi = jnp.where(wid == _NW - 1, _V, lo + _REG)
    is_last = wid == _NW - 1

    def phase(idx_hbm, tabT_hbm, tail_hbm, out_hbm):
        pltpu.sync_copy(idx_hbm, idx_v)

        def bucket(g, cnt):
            r16 = idx_v[pl.ds(g * 16, 16)]
            m = (r16 >= lo) & (r16 < hi)
            mi = m.astype(jnp.int32)
            k = lax.reduce_sum(mi, axes=(0,))
            rank = plsc.cumsum(mi)
            tgt = (cnt + rank - 1) & 31
            plsc.store_scatter(ring_r, [tgt], r16, mask=m)
            plsc.store_scatter(ring_p, [tgt], g * 16 + lanes, mask=m)
            newcnt = cnt + k
            crossed = (newcnt >> 4) > (cnt >> 4)
            base = pl.multiple_of((cnt >> 4) << 4, 16)

            @pl.when(crossed & (((cnt >> 4) & 1) == 0))
            def _():
                list_r[pl.ds(base, 16)] = ring_r[pl.ds(0, 16)]
                list_p[pl.ds(base, 16)] = ring_p[pl.ds(0, 16)]

            @pl.when(crossed & (((cnt >> 4) & 1) == 1))
            def _():
                list_r[pl.ds(base, 16)] = ring_r[pl.ds(16, 16)]
                list_p[pl.ds(base, 16)] = ring_p[pl.ds(16, 16)]

            return newcnt

        cnt = lax.fori_loop(0, _B // 16, bucket, jnp.int32(0), unroll=False)
        base = pl.multiple_of((cnt >> 4) << 4, 16)

        @pl.when(((cnt >> 4) & 1) == 0)
        def _():
            list_r[pl.ds(base, 16)] = ring_r[pl.ds(0, 16)]
            list_p[pl.ds(base, 16)] = ring_p[pl.ds(0, 16)]

        @pl.when(((cnt >> 4) & 1) == 1)
        def _():
            list_r[pl.ds(base, 16)] = ring_r[pl.ds(16, 16)]
            list_p[pl.ds(base, 16)] = ring_p[pl.ds(16, 16)]

        nch = (cnt + 15) >> 4

        dump = jnp.full((16,), _B, jnp.int32)
        stagepos_v[0, :] = dump
        stagepos_v[1, :] = dump
        pltpu.async_copy(stage_v.at[pl.ds(0, 16), :],
                         out_hbm.at[stagepos_v.at[0]], sem_sc)

        def chunk_loop(slab_v, win_lo, win_hi, stagecnt):
            def chunk(g, stagecnt):
                r16 = list_r[pl.ds(g * 16, 16)]
                p16 = list_p[pl.ds(g * 16, 16)]
                m = (r16 >= win_lo) & (r16 < win_hi)
                m = m & ((g * 16 + lanes) < cnt)
                mi = m.astype(jnp.int32)
                k = lax.reduce_sum(mi, axes=(0,))

                @pl.when(k > 0)
                def _():
                    local16 = jnp.where(m, r16 - win_lo, 0)
                    rank = plsc.cumsum(mi)
                    tgt = (stagecnt + rank - 1) & 31
                    for f in range(_D):
                        fv = jnp.full((16,), f, jnp.int32)
                        v = plsc.load_gather(slab_v, [fv, local16])
                        plsc.store_scatter(stage_v, [tgt, fv], v, mask=m)
                    plsc.store_scatter(stagepos_v, [tgt >> 4, tgt & 15],
                                       p16, mask=m)

                newsc = stagecnt + k
                crossed = (newsc >> 4) > (stagecnt >> 4)

                @pl.when(crossed & (((stagecnt >> 4) & 1) == 0))
                def _():
                    pltpu.make_async_copy(
                        stage_v.at[pl.ds(0, 16), :],
                        out_hbm.at[stagepos_v.at[0]], sem_sc).wait()
                    pltpu.async_copy(
                        stage_v.at[pl.ds(0, 16), :],
                        out_hbm.at[stagepos_v.at[0]], sem_sc)

                @pl.when(crossed & (((stagecnt >> 4) & 1) == 1))
                def _():
                    pltpu.make_async_copy(
                        stage_v.at[pl.ds(16, 16), :],
                        out_hbm.at[stagepos_v.at[1]], sem_sc).wait()
                    pltpu.async_copy(
                        stage_v.at[pl.ds(16, 16), :],
                        out_hbm.at[stagepos_v.at[1]], sem_sc)

                return newsc

            return lax.fori_loop(0, nch, chunk, stagecnt, unroll=False)

        first = pl.multiple_of(lo, 128)
        pltpu.async_copy(tabT_hbm.at[:, pl.ds(first, _SLAB)], slab0_v, sem_b0)

        def do_slab(s, stagecnt):
            slab_lo = pl.multiple_of(lo + s * _SLAB, 128)
            nxt = pl.multiple_of(lo + (s + 1) * _SLAB, 128)
            issue_next = s + 1 < _NSLAB_T

            @pl.when(issue_next & ((s & 1) == 0))
            def _():
                pltpu.async_copy(tabT_hbm.at[:, pl.ds(nxt, _SLAB)],
                                 slab1_v, sem_b1)

            @pl.when(issue_next & ((s & 1) == 1))
            def _():
                pltpu.async_copy(tabT_hbm.at[:, pl.ds(nxt, _SLAB)],
                                 slab0_v, sem_b0)

            def even(stagecnt):
                pltpu.make_async_copy(
                    tabT_hbm.at[:, pl.ds(first, _SLAB)], slab0_v,
                    sem_b0).wait()
                return chunk_loop(slab0_v, slab_lo, slab_lo + _SLAB, stagecnt)

            def odd(stagecnt):
                pltpu.make_async_copy(
                    tabT_hbm.at[:, pl.ds(first, _SLAB)], slab1_v,
                    sem_b1).wait()
                return chunk_loop(slab1_v, slab_lo, slab_lo + _SLAB, stagecnt)

            return lax.cond((s & 1) == 0, even, odd, stagecnt)

        stagecnt = lax.fori_loop(0, _NSLAB_T, do_slab, jnp.int32(0),
                                 unroll=False)

        @pl.when(is_last)
        def _():
            pltpu.async_copy(
                tabT_hbm.at[:, pl.ds(_LAST_FULL, _SLAB)], slab0_v,
                sem_b0).wait()

        stagecnt = lax.cond(
            is_last,
            lambda sc: chunk_loop(slab0_v, _LAST_FULL, _TAIL_LO, sc),
            lambda sc: sc, stagecnt)

        @pl.when(is_last)
        def _():
            pltpu.async_copy(tail_hbm, slab1_v.at[:, pl.ds(0, 128)],
                             sem_b1).wait()

        stagecnt = lax.cond(
            is_last,
            lambda sc: chunk_loop(slab1_v, _TAIL_LO, _V, sc),
            lambda sc: sc, stagecnt)

        pltpu.make_async_copy(stage_v.at[pl.ds(0, 16), :],
                              out_hbm.at[stagepos_v.at[0]], sem_sc).wait()
        inhalf = stagecnt & 15
        hh = (stagecnt >> 4) & 1
        plsc.store_scatter(stagepos_v,
                           [jnp.full((16,), hh, jnp.int32), lanes],
                           dump, mask=lanes >= inhalf)

        @pl.when(hh == 0)
        def _():
            pltpu.async_copy(stage_v.at[pl.ds(0, 16), :],
                             out_hbm.at[stagepos_v.at[0]], sem_sc).wait()

        @pl.when(hh == 1)
        def _():
            pltpu.async_copy(stage_v.at[pl.ds(16, 16), :],
                             out_hbm.at[stagepos_v.at[1]], sem_sc).wait()

    phase(users_hbm, utabT_hbm, utail_hbm, uout_hbm)
    phase(items_hbm, itabT_hbm, itail_hbm, iout_hbm)


_BLK = 2048


def _mlp_body(u_ref, i_ref, w1_ref, b1_ref, w2_ref, b2_ref, o_ref):
    u = u_ref[:, :_D]
    i = i_ref[:, :_D]
    w1 = w1_ref[...]
    h = lax.dot_general(u, w1[:, :_D], (((1,), (1,)), ((), ())),
                        preferred_element_type=jnp.float32,
                        precision=lax.Precision.HIGHEST)
    h += lax.dot_general(i, w1[:, _D:], (((1,), (1,)), ((), ())),
                         preferred_element_type=jnp.float32,
                         precision=lax.Precision.HIGHEST)
    h += b1_ref[...]
    h = jnp.maximum(h, 0.0)
    o_ref[...] = jnp.sum(h * w2_ref[...], axis=1) + b2_ref[0, 0]


_mlp = pl.pallas_call(
    _mlp_body,
    grid=(_B // _BLK,),
    in_specs=[
        pl.BlockSpec((_BLK, 2 * _D), lambda b: (b, 0)),
        pl.BlockSpec((_BLK, 2 * _D), lambda b: (b, 0)),
        pl.BlockSpec((_H, 2 * _D), lambda b: (0, 0)),
        pl.BlockSpec((1, _H), lambda b: (0, 0)),
        pl.BlockSpec((1, _H), lambda b: (0, 0)),
        pl.BlockSpec((1, 1), lambda b: (0, 0)),
    ],
    out_specs=pl.BlockSpec((_BLK,), lambda b: (b,)),
    out_shape=jax.ShapeDtypeStruct((_B,), jnp.float32),
)


def kernel(users, items, user_table, item_table, w1, b1, w2, b2):
    users = users.astype(jnp.int32)
    items = items.astype(jnp.int32)
    utabT = user_table.T
    itabT = item_table.T
    utail = user_table[_TAIL_LO:].T
    itail = item_table[_TAIL_LO:].T
    u_emb, i_emb = _sc_gather(users, items, utabT, itabT, utail, itail)
    return _mlp(u_emb, i_emb, w1,
                b1.reshape(1, _H), w2.reshape(1, _H), b2.reshape(1, 1))


# speedup vs baseline: 2.3736x; 1.1178x over previous
import functools

import jax
import jax.numpy as jnp
from jax import lax
from jax.experimental import pallas as pl
from jax.experimental.pallas import tpu as pltpu
from jax.experimental.pallas import tpu_sc as plsc

_B = 16384
_D = 64
_H = 128
_V = 1000000

_info = plsc.get_sparse_core_info()
_NC, _NS = _info.num_cores, _info.num_subcores
_NW = _NC * _NS
_SLAB = 256
_NSLAB_T = (_V // _SLAB) // _NW
_REG = _NSLAB_T * _SLAB
_LAST_FULL = _NW * _REG
_TAIL_LO = _V - 128
_NSUB = 8
_SUBSH = 12
_SPS = (1 << _SUBSH) // _SLAB
_LCAP1 = _B + 16
_LCAP2 = _B + 16 * _NSUB

_mesh = plsc.VectorSubcoreMesh(core_axis_name="c", subcore_axis_name="s")


@functools.partial(
    pl.kernel,
    out_type=[
        jax.ShapeDtypeStruct((_B + 16, 2 * _D), jnp.float32),
        jax.ShapeDtypeStruct((_B + 16, 2 * _D), jnp.float32),
    ],
    mesh=_mesh,
    compiler_params=pltpu.CompilerParams(needs_layout_passes=False),
    scratch_types=[
        pltpu.VMEM((_B,), jnp.int32),
        pltpu.VMEM((_LCAP1,), jnp.int32),
        pltpu.VMEM((_LCAP1,), jnp.int32),
        pltpu.VMEM((_LCAP2,), jnp.int32),
        pltpu.VMEM((_LCAP2,), jnp.int32),
        pltpu.VMEM((32,), jnp.int32),
        pltpu.VMEM((32,), jnp.int32),
        pltpu.VMEM((_D, _SLAB), jnp.float32),
        pltpu.VMEM((_D, _SLAB), jnp.float32),
        pltpu.VMEM((32, 2 * _D), jnp.float32),
        pltpu.VMEM((2, 16), jnp.int32),
        pltpu.SemaphoreType.DMA,
        pltpu.SemaphoreType.DMA,
        pltpu.SemaphoreType.DMA,
    ],
)
def _sc_gather(users_hbm, items_hbm, utabT_hbm, itabT_hbm,
               utail_hbm, itail_hbm, uout_hbm, iout_hbm,
               idx_v, list_r, list_p, list2_r, list2_p, ring_r, ring_p,
               slab0_v, slab1_v, stage_v, stagepos_v,
               sem_b0, sem_b1, sem_sc):
    wid = lax.axis_index("s") * _NC + lax.axis_index("c")
    lanes = lax.iota(jnp.int32, 16)
    lo = wid * _REG
    hi = jnp.where(wid == _NW - 1, _V, lo + _REG)
    is_last = wid == _NW - 1

    def phase(idx_hbm, tabT_hbm, tail_hbm, out_hbm):
        pltpu.sync_copy(idx_hbm, idx_v)

        def bucket(g, cnt):
            r16 = idx_v[pl.ds(g * 16, 16)]
            m = (r16 >= lo) & (r16 < hi)
            mi = m.astype(jnp.int32)
            k = lax.reduce_sum(mi, axes=(0,))
            rank = plsc.cumsum(mi)
            tgt = (cnt + rank - 1) & 31
            plsc.store_scatter(ring_r, [tgt], r16, mask=m)
            plsc.store_scatter(ring_p, [tgt], g * 16 + lanes, mask=m)
            newcnt = cnt + k
            crossed = (newcnt >> 4) > (cnt >> 4)
            base = pl.multiple_of((cnt >> 4) << 4, 16)

            @pl.when(crossed & (((cnt >> 4) & 1) == 0))
            def _():
                list_r[pl.ds(base, 16)] = ring_r[pl.ds(0, 16)]
                list_p[pl.ds(base, 16)] = ring_p[pl.ds(0, 16)]

            @pl.when(crossed & (((cnt >> 4) & 1) == 1))
            def _():
                list_r[pl.ds(base, 16)] = ring_r[pl.ds(16, 16)]
                list_p[pl.ds(base, 16)] = ring_p[pl.ds(16, 16)]

            return newcnt

        cnt = lax.fori_loop(0, _B // 16, bucket, jnp.int32(0), unroll=False)
        base = pl.multiple_of((cnt >> 4) << 4, 16)

        @pl.when(((cnt >> 4) & 1) == 0)
        def _():
            list_r[pl.ds(base, 16)] = ring_r[pl.ds(0, 16)]
            list_p[pl.ds(base, 16)] = ring_p[pl.ds(0, 16)]

        @pl.when(((cnt >> 4) & 1) == 1)
        def _():
            list_r[pl.ds(base, 16)] = ring_r[pl.ds(16, 16)]
            list_p[pl.ds(base, 16)] = ring_p[pl.ds(16, 16)]

        nch1 = (cnt + 15) >> 4

        subcnt = []
        for j in range(_NSUB):
            jlo = lo + (j << _SUBSH)
            jhi = jlo + (1 << _SUBSH)
            if j == _NSUB - 1:
                jhi = jnp.where(is_last, _V, jhi)

            def hist(g, c, jlo=jlo, jhi=jhi):
                r16 = list_r[pl.ds(g * 16, 16)]
                m = (r16 >= jlo) & (r16 < jhi) & ((g * 16 + lanes) < cnt)
                return c + lax.reduce_sum(m.astype(jnp.int32), axes=(0,))

            subcnt.append(lax.fori_loop(0, nch1, hist, jnp.int32(0),
                                        unroll=False))

        suboff = [jnp.int32(0)]
        for j in range(_NSUB - 1):
            suboff.append(suboff[j] + (((subcnt[j] + 15) >> 4) << 4))

        for j in range(_NSUB):
            jlo = lo + (j << _SUBSH)
            jhi = jlo + (1 << _SUBSH)
            if j == _NSUB - 1:
                jhi = jnp.where(is_last, _V, jhi)
            off_j = suboff[j]

            def fill(g, c, jlo=jlo, jhi=jhi, off_j=off_j):
                r16 = list_r[pl.ds(g * 16, 16)]
                p16 = list_p[pl.ds(g * 16, 16)]
                m = (r16 >= jlo) & (r16 < jhi) & ((g * 16 + lanes) < cnt)
                mi = m.astype(jnp.int32)
                k = lax.reduce_sum(mi, axes=(0,))
                rank = plsc.cumsum(mi)
                tgt = (c + rank - 1) & 31
                plsc.store_scatter(ring_r, [tgt], r16, mask=m)
                plsc.store_scatter(ring_p, [tgt], p16, mask=m)
                newc = c + k
                crossed = (newc >> 4) > (c >> 4)
                fbase = pl.multiple_of(off_j + ((c >> 4) << 4), 16)

                @pl.when(crossed & (((c >> 4) & 1) == 0))
                def _():
                    list2_r[pl.ds(fbase, 16)] = ring_r[pl.ds(0, 16)]
                    list2_p[pl.ds(fbase, 16)] = ring_p[pl.ds(0, 16)]

                @pl.when(crossed & (((c >> 4) & 1) == 1))
                def _():
                    list2_r[pl.ds(fbase, 16)] = ring_r[pl.ds(16, 16)]
                    list2_p[pl.ds(fbase, 16)] = ring_p[pl.ds(16, 16)]

                return newc

            c_end = lax.fori_loop(0, nch1, fill, jnp.int32(0), unroll=False)
            fbase = pl.multiple_of(off_j + ((c_end >> 4) << 4), 16)

            @pl.when(((c_end >> 4) & 1) == 0)
            def _():
                list2_r[pl.ds(fbase, 16)] = ring_r[pl.ds(0, 16)]
                list2_p[pl.ds(fbase, 16)] = ring_p[pl.ds(0, 16)]

            @pl.when(((c_end >> 4) & 1) == 1)
            def _():
                list2_r[pl.ds(fbase, 16)] = ring_r[pl.ds(16, 16)]
                list2_p[pl.ds(fbase, 16)] = ring_p[pl.ds(16, 16)]

        def sel8(vals, j):
            r = vals[0]
            for q in range(1, _NSUB):
                r = jnp.where(j == q, vals[q], r)
            return r

        dump = jnp.full((16,), _B, jnp.int32)
        stagepos_v[0, :] = dump
        stagepos_v[1, :] = dump
        pltpu.async_copy(stage_v.at[pl.ds(0, 16), :],
                         out_hbm.at[stagepos_v.at[0]], sem_sc)

        def chunk_loop(slab_v, win_lo, win_hi, off_j, cnt_j, stagecnt):
            def chunk(g, stagecnt):
                gbase = pl.multiple_of(off_j + (g << 4), 16)
                r16 = list2_r[pl.ds(gbase, 16)]
                m = (r16 >= win_lo) & (r16 < win_hi)
                m = m & (((g << 4) + lanes) < cnt_j)
                mi = m.astype(jnp.int32)
                k = lax.reduce_sum(mi, axes=(0,))

                @pl.when(k > 0)
                def _():
                    p16 = list2_p[pl.ds(gbase, 16)]
                    local16 = jnp.where(m, r16 - win_lo, 0)
                    rank = plsc.cumsum(mi)
                    tgt = (stagecnt + rank - 1) & 31
                    for f in range(_D):
                        fv = jnp.full((16,), f, jnp.int32)
                        v = plsc.load_gather(slab_v, [fv, local16])
                        plsc.store_scatter(stage_v, [tgt, fv], v, mask=m)
                    plsc.store_scatter(stagepos_v, [tgt >> 4, tgt & 15],
                                       p16, mask=m)

                newsc = stagecnt + k
                crossed = (newsc >> 4) > (stagecnt >> 4)

                @pl.when(crossed & (((stagecnt >> 4) & 1) == 0))
                def _():
                    pltpu.make_async_copy(
                        stage_v.at[pl.ds(0, 16), :],
                        out_hbm.at[stagepos_v.at[0]], sem_sc).wait()
                    pltpu.async_copy(
                        stage_v.at[pl.ds(0, 16), :],
                        out_hbm.at[stagepos_v.at[0]], sem_sc)

                @pl.when(crossed & (((stagecnt >> 4) & 1) == 1))
                def _():
                    pltpu.make_async_copy(
                        stage_v.at[pl.ds(16, 16), :],
                        out_hbm.at[stagepos_v.at[1]], sem_sc).wait()
                    pltpu.async_copy(
                        stage_v.at[pl.ds(16, 16), :],
                        out_hbm.at[stagepos_v.at[1]], sem_sc)

                return newsc

            nch = (cnt_j + 15) >> 4
            return lax.fori_loop(0, nch, chunk, stagecnt, unroll=False)

        first = pl.multiple_of(lo, 128)
        pltpu.async_copy(tabT_hbm.at[:, pl.ds(first, _SLAB)], slab0_v, sem_b0)

        def do_slab(s, stagecnt):
            slab_lo = pl.multiple_of(lo + s * _SLAB, 128)
            nxt = pl.multiple_of(lo + (s + 1) * _SLAB, 128)
            issue_next = s + 1 < _NSLAB_T

            @pl.when(issue_next & ((s & 1) == 0))
            def _():
                pltpu.async_copy(tabT_hbm.at[:, pl.ds(nxt, _SLAB)],
                                 slab1_v, sem_b1)

            @pl.when(issue_next & ((s & 1) == 1))
            def _():
                pltpu.async_copy(tabT_hbm.at[:, pl.ds(nxt, _SLAB)],
                                 slab0_v, sem_b0)

            j = s // _SPS
            off_j = sel8(suboff, j)
            cnt_j = sel8(subcnt, j)

            def even(stagecnt):
                pltpu.make_async_copy(
                    tabT_hbm.at[:, pl.ds(first, _SLAB)], slab0_v,
                    sem_b0).wait()
                return chunk_loop(slab0_v, slab_lo, slab_lo + _SLAB,
                                  off_j, cnt_j, stagecnt)

            def odd(stagecnt):
                pltpu.make_async_copy(
                    tabT_hbm.at[:, pl.ds(first, _SLAB)], slab1_v,
                    sem_b1).wait()
                return chunk_loop(slab1_v, slab_lo, slab_lo + _SLAB,
                                  off_j, cnt_j, stagecnt)

            return lax.cond((s & 1) == 0, even, odd, stagecnt)

        stagecnt = lax.fori_loop(0, _NSLAB_T, do_slab, jnp.int32(0),
                                 unroll=False)

        off_7 = suboff[_NSUB - 1]
        cnt_7 = subcnt[_NSUB - 1]

        for e, elo in enumerate((_LAST_FULL, _LAST_FULL + _SLAB)):
            win_hi = min(elo + _SLAB, _TAIL_LO)

            @pl.when(is_last)
            def _(elo=elo):
                pltpu.async_copy(
                    tabT_hbm.at[:, pl.ds(elo, _SLAB)], slab0_v,
                    sem_b0).wait()

            stagecnt = lax.cond(
                is_last,
                lambda sc, elo=elo, win_hi=win_hi: chunk_loop(
                    slab0_v, elo, win_hi, off_7, cnt_7, sc),
                lambda sc: sc, stagecnt)

        @pl.when(is_last)
        def _():
            pltpu.async_copy(tail_hbm, slab1_v.at[:, pl.ds(0, 128)],
                             sem_b1).wait()

        stagecnt = lax.cond(
            is_last,
            lambda sc: chunk_loop(slab1_v, _TAIL_LO, _V, off_7, cnt_7, sc),
            lambda sc: sc, stagecnt)

        pltpu.make_async_copy(stage_v.at[pl.ds(0, 16), :],
                              out_hbm.at[stagepos_v.at[0]], sem_sc).wait()
        inhalf = stagecnt & 15
        hh = (stagecnt >> 4) & 1
        plsc.store_scatter(stagepos_v,
                           [jnp.full((16,), hh, jnp.int32), lanes],
                           dump, mask=lanes >= inhalf)

        @pl.when(hh == 0)
        def _():
            pltpu.async_copy(stage_v.at[pl.ds(0, 16), :],
                             out_hbm.at[stagepos_v.at[0]], sem_sc).wait()

        @pl.when(hh == 1)
        def _():
            pltpu.async_copy(stage_v.at[pl.ds(16, 16), :],
                             out_hbm.at[stagepos_v.at[1]], sem_sc).wait()

    phase(users_hbm, utabT_hbm, utail_hbm, uout_hbm)
    phase(items_hbm, itabT_hbm, itail_hbm, iout_hbm)


_BLK = 2048


def _mlp_body(u_ref, i_ref, w1_ref, b1_ref, w2_ref, b2_ref, o_ref):
    u = u_ref[:, :_D]
    i = i_ref[:, :_D]
    w1 = w1_ref[...]
    h = lax.dot_general(u, w1[:, :_D], (((1,), (1,)), ((), ())),
                        preferred_element_type=jnp.float32,
                        precision=lax.Precision.HIGHEST)
    h += lax.dot_general(i, w1[:, _D:], (((1,), (1,)), ((), ())),
                         preferred_element_type=jnp.float32,
                         precision=lax.Precision.HIGHEST)
    h += b1_ref[...]
    h = jnp.maximum(h, 0.0)
    o_ref[...] = jnp.sum(h * w2_ref[...], axis=1) + b2_ref[0, 0]


_mlp = pl.pallas_call(
    _mlp_body,
    grid=(_B // _BLK,),
    in_specs=[
        pl.BlockSpec((_BLK, 2 * _D), lambda b: (b, 0)),
        pl.BlockSpec((_BLK, 2 * _D), lambda b: (b, 0)),
        pl.BlockSpec((_H, 2 * _D), lambda b: (0, 0)),
        pl.BlockSpec((1, _H), lambda b: (0, 0)),
        pl.BlockSpec((1, _H), lambda b: (0, 0)),
        pl.BlockSpec((1, 1), lambda b: (0, 0)),
    ],
    out_specs=pl.BlockSpec((_BLK,), lambda b: (b,)),
    out_shape=jax.ShapeDtypeStruct((_B,), jnp.float32),
)


def kernel(users, items, user_table, item_table, w1, b1, w2, b2):
    users = users.astype(jnp.int32)
    items = items.astype(jnp.int32)
    utabT = user_table.T
    itabT = item_table.T
    utail = user_table[_TAIL_LO:].T
    itail = item_table[_TAIL_LO:].T
    u_emb, i_emb = _sc_gather(users, items, utabT, itabT, utail, itail)
    return _mlp(u_emb, i_emb, w1,
                b1.reshape(1, _H), w2.reshape(1, _H), b2.reshape(1, 1))
